# baseline (speedup 1.0000x reference)
import functools

import jax
import jax.numpy as jnp
from jax import lax
from jax.experimental import pallas as pl
from jax.experimental.pallas import tpu as pltpu
from jax.experimental.pallas import tpu_sc as plsc

_EMBED_DIM = 64
_GROUP = 128
_NGATHER = 4
_NSTAGE = 2


@functools.cache
def _build_gather(batch: int, seq: int):
    try:
        info = plsc.get_sparse_core_info()
        num_cores, num_subcores = info.num_cores, info.num_subcores
    except Exception:
        num_cores, num_subcores = 2, 16
    num_workers = num_cores * num_subcores
    blocks = batch // _GROUP
    num_groups = seq * blocks
    assert batch % _GROUP == 0 and num_groups % num_workers == 0
    gpw = num_groups // num_workers
    assert gpw % _NGATHER == 0 and gpw // _NGATHER >= 2
    ipw = gpw * _GROUP

    mesh = plsc.VectorSubcoreMesh(core_axis_name="c", subcore_axis_name="s")

    @functools.partial(
        pl.kernel,
        out_type=jax.ShapeDtypeStruct((seq, _EMBED_DIM, batch), jnp.float32),
        mesh=mesh,
        scratch_types=(
            [pltpu.VMEM((ipw,), jnp.int32)]
            + [pltpu.VMEM((_GROUP, _EMBED_DIM), jnp.float32) for _ in range(_NGATHER)]
            + [pltpu.VMEM((_EMBED_DIM, _GROUP), jnp.float32) for _ in range(_NSTAGE)]
            + [pltpu.SemaphoreType.DMA for _ in range(_NGATHER + _NSTAGE)]
        ),
        compiler_params=pltpu.CompilerParams(
            use_tc_tiling_on_sc=False, needs_layout_passes=False
        ),
    )
    def gather(table_hbm, idx_hbm, out_hbm, *scr):
        idx_v = scr[0]
        rows_v = scr[1 : 1 + _NGATHER]
        staged_v = scr[1 + _NGATHER : 1 + _NGATHER + _NSTAGE]
        sem_g = scr[1 + _NGATHER + _NSTAGE : 1 + 2 * _NGATHER + _NSTAGE]
        sem_w = scr[1 + 2 * _NGATHER + _NSTAGE :]

        wid = lax.axis_index("s") * num_cores + lax.axis_index("c")
        g0 = wid * gpw

        iota = lax.iota(jnp.int32, 16)
        zeros = iota * 0
        bases = [iota + cb * 16 for cb in range(8)]

        def fire_gather(g, rs):
            pltpu.async_copy(
                table_hbm.at[idx_v.at[pl.ds(g * _GROUP, _GROUP)]],
                rows_v[rs],
                sem_g[rs],
            )

        def wait_gather(g, rs):
            pltpu.make_async_copy(
                table_hbm.at[idx_v.at[pl.ds(g * _GROUP, _GROUP)]],
                rows_v[rs],
                sem_g[rs],
            ).wait()

        def out_slice(g):
            gg = g0 + g
            s = gg // blocks
            k = gg % blocks
            return out_hbm.at[s, :, pl.ds(k * _GROUP, _GROUP)]

        def fire_writeout(g, ws):
            pltpu.async_copy(staged_v[ws], out_slice(g), sem_w[ws])

        def wait_writeout(g, ws):
            pltpu.make_async_copy(staged_v[ws], out_slice(g), sem_w[ws]).wait()

        def transpose(rs, ws):
            src = rows_v[rs]
            dst = staged_v[ws]

            def col(j, carry):
                jv = zeros + j
                for cb in range(8):
                    vec = plsc.load_gather(src, [bases[cb], jv])
                    dst[j, pl.ds(cb * 16, 16)] = vec
                return carry

            lax.fori_loop(0, _EMBED_DIM, col, 0)

        pltpu.sync_copy(idx_hbm.at[pl.ds(wid * ipw, ipw)], idx_v)

        for g in range(_NGATHER):
            fire_gather(g, g)

        for b in range(_NGATHER):
            g = b
            ws = b % _NSTAGE
            if g >= _NSTAGE:
                wait_writeout(g - _NSTAGE, ws)
            wait_gather(g, b)
            transpose(b, ws)
            fire_writeout(g, ws)
            fire_gather(g + _NGATHER, b)

        def block(jb, carry):
            for b in range(_NGATHER):
                g = jb * _NGATHER + b
                ws = b % _NSTAGE
                wait_writeout(g - _NSTAGE, ws)
                wait_gather(g, b)
                transpose(b, ws)
                fire_writeout(g, ws)
                fire_gather(g + _NGATHER, b)
            return carry

        lax.fori_loop(1, gpw // _NGATHER - 1, block, 0)

        for b in range(_NGATHER):
            g = gpw - _NGATHER + b
            ws = b % _NSTAGE
            wait_writeout(g - _NSTAGE, ws)
            wait_gather(g, b)
            transpose(b, ws)
            fire_writeout(g, ws)

        for b in range(_NSTAGE):
            g = gpw - _NSTAGE + b
            wait_writeout(g, g % _NSTAGE)

    return gather


def kernel(token_ids, weight):
    batch, seq = token_ids.shape
    idx = jnp.transpose(token_ids).reshape(-1).astype(jnp.int32)
    out = _build_gather(batch, seq)(weight, idx)
    return jnp.transpose(out, (2, 0, 1))


# speedup vs baseline: 1.3545x; 1.3545x over previous
import functools

import jax
import jax.numpy as jnp
from jax import lax
from jax.experimental import pallas as pl
from jax.experimental.pallas import tpu as pltpu
from jax.experimental.pallas import tpu_sc as plsc

_EMBED_DIM = 64
_GROUP = 128
_NGATHER = 4
_NSTAGE = 2


@functools.cache
def _build_gather(batch: int, seq: int):
    try:
        info = plsc.get_sparse_core_info()
        num_cores, num_subcores = info.num_cores, info.num_subcores
    except Exception:
        num_cores, num_subcores = 2, 16
    num_workers = num_cores * num_subcores
    blocks = batch // _GROUP
    num_groups = seq * blocks
    assert batch % _GROUP == 0 and num_groups % num_workers == 0
    gpw = num_groups // num_workers
    assert gpw % _NGATHER == 0 and gpw // _NGATHER >= 2
    ipw = gpw * _GROUP

    mesh = plsc.VectorSubcoreMesh(core_axis_name="c", subcore_axis_name="s")

    @functools.partial(
        pl.kernel,
        out_type=jax.ShapeDtypeStruct((seq, _EMBED_DIM, batch), jnp.float32),
        mesh=mesh,
        scratch_types=(
            [pltpu.VMEM((ipw,), jnp.int32)]
            + [pltpu.VMEM((_GROUP, _EMBED_DIM), jnp.float32) for _ in range(_NGATHER)]
            + [pltpu.VMEM((_EMBED_DIM, _GROUP), jnp.float32) for _ in range(_NSTAGE)]
            + [pltpu.SemaphoreType.DMA for _ in range(_NGATHER + _NSTAGE)]
        ),
        compiler_params=pltpu.CompilerParams(
            use_tc_tiling_on_sc=False, needs_layout_passes=False
        ),
    )
    def gather(table_hbm, idx_hbm, out_hbm, *scr):
        idx_v = scr[0]
        rows_v = scr[1 : 1 + _NGATHER]
        staged_v = scr[1 + _NGATHER : 1 + _NGATHER + _NSTAGE]
        sem_g = scr[1 + _NGATHER + _NSTAGE : 1 + 2 * _NGATHER + _NSTAGE]
        sem_w = scr[1 + 2 * _NGATHER + _NSTAGE :]

        wid = lax.axis_index("s") * num_cores + lax.axis_index("c")
        g0 = wid * gpw

        iota = lax.iota(jnp.int32, 16)
        zeros = iota * 0
        bases = [iota + cb * 16 for cb in range(8)]

        def fire_gather(g, rs):
            pltpu.async_copy(
                table_hbm.at[idx_v.at[pl.ds(g * _GROUP, _GROUP)]],
                rows_v[rs],
                sem_g[rs],
            )

        def wait_gather(g, rs):
            pltpu.make_async_copy(
                table_hbm.at[idx_v.at[pl.ds(g * _GROUP, _GROUP)]],
                rows_v[rs],
                sem_g[rs],
            ).wait()

        def out_slice(g):
            gg = g0 + g
            s = gg // blocks
            k = gg % blocks
            return out_hbm.at[s, :, pl.ds(k * _GROUP, _GROUP)]

        def fire_writeout(g, ws):
            pltpu.async_copy(staged_v[ws], out_slice(g), sem_w[ws])

        def wait_writeout(g, ws):
            pltpu.make_async_copy(staged_v[ws], out_slice(g), sem_w[ws]).wait()

        def transpose(rs, ws):
            src = rows_v[rs]
            dst = staged_v[ws]

            @plsc.parallel_loop(0, _EMBED_DIM, step=1, unroll=8)
            def col(j):
                jv = zeros + j
                for cb in range(8):
                    vec = plsc.load_gather(src, [bases[cb], jv])
                    dst[j, pl.ds(cb * 16, 16)] = vec

        pltpu.sync_copy(idx_hbm.at[pl.ds(wid * ipw, ipw)], idx_v)

        for g in range(_NGATHER):
            fire_gather(g, g)

        for b in range(_NGATHER):
            g = b
            ws = b % _NSTAGE
            if g >= _NSTAGE:
                wait_writeout(g - _NSTAGE, ws)
            wait_gather(g, b)
            transpose(b, ws)
            fire_writeout(g, ws)
            fire_gather(g + _NGATHER, b)

        def block(jb, carry):
            for b in range(_NGATHER):
                g = jb * _NGATHER + b
                ws = b % _NSTAGE
                wait_writeout(g - _NSTAGE, ws)
                wait_gather(g, b)
                transpose(b, ws)
                fire_writeout(g, ws)
                fire_gather(g + _NGATHER, b)
            return carry

        lax.fori_loop(1, gpw // _NGATHER - 1, block, 0)

        for b in range(_NGATHER):
            g = gpw - _NGATHER + b
            ws = b % _NSTAGE
            wait_writeout(g - _NSTAGE, ws)
            wait_gather(g, b)
            transpose(b, ws)
            fire_writeout(g, ws)

        for b in range(_NSTAGE):
            g = gpw - _NSTAGE + b
            wait_writeout(g, g % _NSTAGE)

    return gather


def kernel(token_ids, weight):
    batch, seq = token_ids.shape
    idx = jnp.transpose(token_ids).reshape(-1).astype(jnp.int32)
    out = _build_gather(batch, seq)(weight, idx)
    return jnp.transpose(out, (2, 0, 1))


# speedup vs baseline: 2.1019x; 1.5518x over previous
import functools

import jax
import jax.numpy as jnp
from jax import lax
from jax.experimental import pallas as pl
from jax.experimental.pallas import tpu as pltpu
from jax.experimental.pallas import tpu_sc as plsc

_EMBED_DIM = 64
_GROUP = 128
_NGATHER = 4
_NSTAGE = 2


@functools.cache
def _build_gather(batch: int, seq: int):
    try:
        info = plsc.get_sparse_core_info()
        num_cores, num_subcores = info.num_cores, info.num_subcores
    except Exception:
        num_cores, num_subcores = 2, 16
    num_workers = num_cores * num_subcores
    blocks = batch // _GROUP
    num_groups = seq * blocks
    assert batch % _GROUP == 0 and num_groups % num_workers == 0
    gpw = num_groups // num_workers
    assert gpw % _NGATHER == 0 and gpw // _NGATHER >= 2
    ipw = gpw * _GROUP

    mesh = plsc.VectorSubcoreMesh(core_axis_name="c", subcore_axis_name="s")

    @functools.partial(
        pl.kernel,
        out_type=jax.ShapeDtypeStruct((seq, _EMBED_DIM, batch), jnp.float32),
        mesh=mesh,
        scratch_types=(
            [pltpu.VMEM((ipw,), jnp.int32)]
            + [pltpu.VMEM((_GROUP, _EMBED_DIM), jnp.float32) for _ in range(_NGATHER)]
            + [pltpu.VMEM((_EMBED_DIM, _GROUP), jnp.float32) for _ in range(_NSTAGE)]
            + [pltpu.SemaphoreType.DMA for _ in range(_NGATHER + _NSTAGE)]
        ),
        compiler_params=pltpu.CompilerParams(
            use_tc_tiling_on_sc=False, needs_layout_passes=False
        ),
    )
    def gather(table_hbm, idx_hbm, out_hbm, *scr):
        idx_v = scr[0]
        rows_v = scr[1 : 1 + _NGATHER]
        staged_v = scr[1 + _NGATHER : 1 + _NGATHER + _NSTAGE]
        sem_g = scr[1 + _NGATHER + _NSTAGE : 1 + 2 * _NGATHER + _NSTAGE]
        sem_w = scr[1 + 2 * _NGATHER + _NSTAGE :]

        wid = lax.axis_index("s") * num_cores + lax.axis_index("c")
        g0 = wid * gpw

        iota = lax.iota(jnp.int32, 16)

        def fire_gather(g, rs):
            pltpu.async_copy(
                table_hbm.at[idx_v.at[pl.ds(g * _GROUP, _GROUP)]],
                rows_v[rs],
                sem_g[rs],
            )

        def wait_gather(g, rs):
            pltpu.make_async_copy(
                table_hbm.at[idx_v.at[pl.ds(g * _GROUP, _GROUP)]],
                rows_v[rs],
                sem_g[rs],
            ).wait()

        def out_slice(g):
            gg = g0 + g
            s = gg // blocks
            k = gg % blocks
            return out_hbm.at[s, :, pl.ds(k * _GROUP, _GROUP)]

        def fire_writeout(g, ws):
            pltpu.async_copy(staged_v[ws], out_slice(g), sem_w[ws])

        def wait_writeout(g, ws):
            pltpu.make_async_copy(staged_v[ws], out_slice(g), sem_w[ws]).wait()

        def transpose(rs, ws):
            src = rows_v[rs]
            dst = staged_v[ws]

            @plsc.parallel_loop(0, 32, step=1, unroll=2)
            def blk(m):
                j0 = (m // 8) * 16
                c0 = (m % 8) * 16
                cvec = iota + c0
                for d in range(16):
                    rot = (iota + d) & 15
                    jvec = rot + j0
                    vec = plsc.load_gather(src, [cvec, jvec])
                    plsc.store_scatter(dst, [jvec, cvec], vec)

        pltpu.sync_copy(idx_hbm.at[pl.ds(wid * ipw, ipw)], idx_v)

        for g in range(_NGATHER):
            fire_gather(g, g)

        for b in range(_NGATHER):
            g = b
            ws = b % _NSTAGE
            if g >= _NSTAGE:
                wait_writeout(g - _NSTAGE, ws)
            wait_gather(g, b)
            transpose(b, ws)
            fire_writeout(g, ws)
            fire_gather(g + _NGATHER, b)

        def block(jb, carry):
            for b in range(_NGATHER):
                g = jb * _NGATHER + b
                ws = b % _NSTAGE
                wait_writeout(g - _NSTAGE, ws)
                wait_gather(g, b)
                transpose(b, ws)
                fire_writeout(g, ws)
                fire_gather(g + _NGATHER, b)
            return carry

        lax.fori_loop(1, gpw // _NGATHER - 1, block, 0)

        for b in range(_NGATHER):
            g = gpw - _NGATHER + b
            ws = b % _NSTAGE
            wait_writeout(g - _NSTAGE, ws)
            wait_gather(g, b)
            transpose(b, ws)
            fire_writeout(g, ws)

        for b in range(_NSTAGE):
            g = gpw - _NSTAGE + b
            wait_writeout(g, g % _NSTAGE)

    return gather


def kernel(token_ids, weight):
    batch, seq = token_ids.shape
    idx = jnp.transpose(token_ids).reshape(-1).astype(jnp.int32)
    out = _build_gather(batch, seq)(weight, idx)
    return jnp.transpose(out, (2, 0, 1))
